# baseline (speedup 1.0000x reference)
import functools

import jax
import jax.numpy as jnp
from jax import lax
from jax.experimental import pallas as pl
from jax.experimental.pallas import tpu as pltpu
from jax.experimental.pallas import tpu_sc as plsc

N = 10000
E = 320000
F = 128
PW = 16
NC, NS = 2, 16
NW = NC * NS
EPT = E // NW
CH = 128
NCH = 80
EPAD = NCH * CH - EPT
NR = N + 8
ZCH = 100
NZ = N // ZCH
ZPT = -(-NZ // NS)


def _mesh():
    return plsc.VectorSubcoreMesh(
        core_axis_name="c", subcore_axis_name="s", num_cores=NC, num_subcores=NS
    )


def _pipelined_agg(tab_hbm, eidx_hbm, wid, rows, ebs, acc_sh,
                   gsems, ssems, isems, extra_scatter=None, extra_wait=None):
    ebase = wid * (NCH * 2)

    for b in (0, 1):
        pltpu.sync_copy(eidx_hbm.at[pl.ds(ebase + 2 * b, 2)], ebs[b])
        pltpu.async_copy(tab_hbm.at[ebs[b].at[0]], rows[b], gsems[b])

    def step(g, carry):
        for b in (0, 1, 2, 3):
            c = 4 * g + b
            s = b % 2
            q2 = (b + 2) % 4
            pltpu.make_async_copy(tab_hbm.at[ebs[b].at[0]], rows[s],
                                  gsems[s]).wait()
            pltpu.async_copy(rows[s], acc_sh.at[ebs[b].at[1]], ssems[s],
                             add=True)
            if extra_scatter is not None:
                extra_scatter(s, ebs[b].at[1])

            @pl.when(c + 2 < NCH)
            def _():
                pltpu.async_copy(eidx_hbm.at[pl.ds(ebase + 2 * (c + 2), 2)],
                                 ebs[q2], isems[q2])

            pltpu.make_async_copy(rows[s], acc_sh.at[ebs[b].at[1]],
                                  ssems[s]).wait()
            if extra_wait is not None:
                extra_wait(s, ebs[b].at[1])

            @pl.when(c + 2 < NCH)
            def _():
                pltpu.make_async_copy(eidx_hbm.at[pl.ds(ebase + 2 * (c + 2),
                                                        2)],
                                      ebs[q2], isems[q2]).wait()
                pltpu.async_copy(tab_hbm.at[ebs[q2].at[0]], rows[s], gsems[s])

        return carry

    lax.fori_loop(0, NCH // 4, step, 0)


@functools.cache
def _sc_agg_wide():
    return pl.kernel(
        _sc_agg_wide_body,
        out_type=(
            jax.ShapeDtypeStruct((NC, N, F), jnp.float32),
            jax.ShapeDtypeStruct((NC, N, PW), jnp.float32),
        ),
        mesh=_mesh(),
        scratch_types=[
            pltpu.VMEM((2, CH), jnp.int32),
            pltpu.VMEM((2, CH), jnp.int32),
            pltpu.VMEM((2, CH), jnp.int32),
            pltpu.VMEM((2, CH), jnp.int32),
            pltpu.VMEM((CH, F), jnp.float32),
            pltpu.VMEM((CH, F), jnp.float32),
            pltpu.VMEM((CH, PW), jnp.float32),
            pltpu.VMEM((ZCH, PW), jnp.float32),
            pltpu.VMEM_SHARED((NR, F), jnp.float32),
            pltpu.VMEM_SHARED((NR, PW), jnp.float32),
            pltpu.SemaphoreType.DMA,
            pltpu.SemaphoreType.DMA,
            pltpu.SemaphoreType.DMA,
            pltpu.SemaphoreType.DMA,
            pltpu.SemaphoreType.DMA,
            pltpu.SemaphoreType.DMA,
            pltpu.SemaphoreType.DMA,
            pltpu.SemaphoreType.DMA,
            pltpu.SemaphoreType.DMA,
            pltpu.SemaphoreType.DMA,
        ],
        compiler_params=pltpu.CompilerParams(use_tc_tiling_on_sc=False),
    )


def _sc_agg_wide_body(x_hbm, eidx_hbm, ones_hbm, z128_hbm, z16_hbm,
                      acc_out, cnt_out,
                      eb0, eb1, eb2, eb3, rows0, rows1, ones_v, cb_v,
                      acc_sh, cnt_sh,
                      gsem0, gsem1, ssem0, ssem1,
                      isem0, isem1, isem2, isem3, osem0, osem1):
    cid = lax.axis_index("c")
    sid = lax.axis_index("s")
    wid = sid * NC + cid

    pltpu.sync_copy(z128_hbm, rows0.at[pl.ds(0, ZCH)])
    pltpu.sync_copy(z16_hbm, cb_v)
    pltpu.sync_copy(ones_hbm, ones_v)

    def zero_step(j, carry):
        k = sid + NS * j

        @pl.when(k < NZ)
        def _():
            pltpu.sync_copy(rows0.at[pl.ds(0, ZCH)],
                            acc_sh.at[pl.ds(k * ZCH, ZCH)])
            pltpu.sync_copy(cb_v, cnt_sh.at[pl.ds(k * ZCH, ZCH)])

        return carry

    lax.fori_loop(0, ZPT, zero_step, 0)
    plsc.subcore_barrier()
    osems = (osem0, osem1)

    def ones_scatter(s, dst_idx):
        pltpu.async_copy(ones_v, cnt_sh.at[dst_idx], osems[s], add=True)

    def ones_wait(s, dst_idx):
        pltpu.make_async_copy(ones_v, cnt_sh.at[dst_idx], osems[s]).wait()

    _pipelined_agg(x_hbm, eidx_hbm, wid, (rows0, rows1),
                   (eb0, eb1, eb2, eb3), acc_sh,
                   (gsem0, gsem1), (ssem0, ssem1),
                   (isem0, isem1, isem2, isem3),
                   extra_scatter=ones_scatter, extra_wait=ones_wait)
    plsc.subcore_barrier()

    def rb_step(j, carry):
        k = sid + NS * j

        @pl.when(k < NZ)
        def _():
            pltpu.sync_copy(acc_sh.at[pl.ds(k * ZCH, ZCH)],
                            rows0.at[pl.ds(0, ZCH)])
            pltpu.sync_copy(rows0.at[pl.ds(0, ZCH)],
                            acc_out.at[cid, pl.ds(k * ZCH, ZCH)])
            pltpu.sync_copy(cnt_sh.at[pl.ds(k * ZCH, ZCH)], cb_v)
            pltpu.sync_copy(cb_v, cnt_out.at[cid, pl.ds(k * ZCH, ZCH)])

        return carry

    lax.fori_loop(0, ZPT, rb_step, 0)


@functools.cache
def _sc_agg_narrow():
    return pl.kernel(
        _sc_agg_narrow_body,
        out_type=jax.ShapeDtypeStruct((NC, N, PW), jnp.float32),
        mesh=_mesh(),
        scratch_types=[
            pltpu.VMEM((2, CH), jnp.int32),
            pltpu.VMEM((2, CH), jnp.int32),
            pltpu.VMEM((2, CH), jnp.int32),
            pltpu.VMEM((2, CH), jnp.int32),
            pltpu.VMEM((CH, PW), jnp.float32),
            pltpu.VMEM((CH, PW), jnp.float32),
            pltpu.VMEM((ZCH, PW), jnp.float32),
            pltpu.VMEM_SHARED((NR, PW), jnp.float32),
            pltpu.SemaphoreType.DMA,
            pltpu.SemaphoreType.DMA,
            pltpu.SemaphoreType.DMA,
            pltpu.SemaphoreType.DMA,
            pltpu.SemaphoreType.DMA,
            pltpu.SemaphoreType.DMA,
            pltpu.SemaphoreType.DMA,
            pltpu.SemaphoreType.DMA,
        ],
        compiler_params=pltpu.CompilerParams(use_tc_tiling_on_sc=False),
    )


def _sc_agg_narrow_body(p_hbm, eidx_hbm, z16_hbm, acc_out,
                        eb0, eb1, eb2, eb3, rows0, rows1, cb_v, acc_sh,
                        gsem0, gsem1, ssem0, ssem1,
                        isem0, isem1, isem2, isem3):
    cid = lax.axis_index("c")
    sid = lax.axis_index("s")
    wid = sid * NC + cid

    pltpu.sync_copy(z16_hbm, cb_v)

    def zero_step(j, carry):
        k = sid + NS * j

        @pl.when(k < NZ)
        def _():
            pltpu.sync_copy(cb_v, acc_sh.at[pl.ds(k * ZCH, ZCH)])

        return carry

    lax.fori_loop(0, ZPT, zero_step, 0)
    plsc.subcore_barrier()
    _pipelined_agg(p_hbm, eidx_hbm, wid, (rows0, rows1),
                   (eb0, eb1, eb2, eb3), acc_sh,
                   (gsem0, gsem1), (ssem0, ssem1),
                   (isem0, isem1, isem2, isem3))
    plsc.subcore_barrier()

    def rb_step(j, carry):
        k = sid + NS * j

        @pl.when(k < NZ)
        def _():
            pltpu.sync_copy(acc_sh.at[pl.ds(k * ZCH, ZCH)], cb_v)
            pltpu.sync_copy(cb_v, acc_out.at[cid, pl.ds(k * ZCH, ZCH)])

        return carry

    lax.fori_loop(0, ZPT, rb_step, 0)


RBLK = 1000


def _tc1_body(accp, cntp, x, w1l, b1l, w1r, w2, h_out, p_out, cnt_out):
    a = accp[...]
    c = cntp[...]
    acc = a[0] + a[1]
    cnt = jnp.maximum(c[0, :, 0:1] + c[1, :, 0:1], 1.0)
    agg = acc / cnt
    dn = (((1,), (1,)), ((), ()))
    h = (
        lax.dot_general(agg, w1l[...], dn, preferred_element_type=jnp.float32)
        + b1l[...]
        + lax.dot_general(x[...], w1r[...], dn,
                          preferred_element_type=jnp.float32)
    )
    h = jnp.maximum(h, 0.0)
    h_out[...] = h
    p_out[...] = lax.dot_general(h, w2[...], dn,
                                 preferred_element_type=jnp.float32)
    cnt_out[...] = cnt


def _tc1(accp, cntp, x, w1l, b1l, w1r, w2, interpret=False):
    return pl.pallas_call(
        _tc1_body,
        grid=(N // RBLK,),
        in_specs=[
            pl.BlockSpec((NC, RBLK, F), lambda i: (0, i, 0)),
            pl.BlockSpec((NC, RBLK, PW), lambda i: (0, i, 0)),
            pl.BlockSpec((RBLK, F), lambda i: (i, 0)),
            pl.BlockSpec((F, F), lambda i: (0, 0)),
            pl.BlockSpec((1, F), lambda i: (0, 0)),
            pl.BlockSpec((F, F), lambda i: (0, 0)),
            pl.BlockSpec((PW, F), lambda i: (0, 0)),
        ],
        out_specs=[
            pl.BlockSpec((RBLK, F), lambda i: (i, 0)),
            pl.BlockSpec((RBLK, PW), lambda i: (i, 0)),
            pl.BlockSpec((RBLK, 1), lambda i: (i, 0)),
        ],
        out_shape=[
            jax.ShapeDtypeStruct((N, F), jnp.float32),
            jax.ShapeDtypeStruct((N, PW), jnp.float32),
            jax.ShapeDtypeStruct((N, 1), jnp.float32),
        ],
        interpret=interpret,
    )(accp, cntp, x, w1l, b1l, w1r, w2)


def _tc2_body(acc2p, cnt, p, b2l, out_ref):
    a = acc2p[...]
    s = a[0, :, 0:1] + a[1, :, 0:1]
    z = s / cnt[...] + b2l[0, 0] + p[:, 1:2]
    out_ref[...] = jax.nn.sigmoid(z)


def _tc2(acc2p, cnt, p, b2l, interpret=False):
    return pl.pallas_call(
        _tc2_body,
        grid=(N // RBLK,),
        in_specs=[
            pl.BlockSpec((NC, RBLK, PW), lambda i: (0, i, 0)),
            pl.BlockSpec((RBLK, 1), lambda i: (i, 0)),
            pl.BlockSpec((RBLK, PW), lambda i: (i, 0)),
            pl.BlockSpec((1, 1), lambda i: (0, 0)),
        ],
        out_specs=pl.BlockSpec((RBLK, 1), lambda i: (i, 0)),
        out_shape=jax.ShapeDtypeStruct((N, 1), jnp.float32),
        interpret=interpret,
    )(acc2p, cnt, p, b2l)


def kernel(x, edge_index, W1l, b1l, W1r, W2l, b2l, W2r):
    ei = edge_index.astype(jnp.int32)
    srcp = jnp.pad(ei[0].reshape(NW, EPT), ((0, 0), (0, EPAD)),
                   constant_values=0)
    dstp = jnp.pad(ei[1].reshape(NW, EPT), ((0, 0), (0, EPAD)),
                   constant_values=N)
    eidx = jnp.stack([srcp.reshape(NW, NCH, CH), dstp.reshape(NW, NCH, CH)],
                     axis=2).reshape(NW * NCH * 2, CH)
    ones = jnp.ones((CH, PW), jnp.float32)
    z128 = jnp.zeros((ZCH, F), jnp.float32)
    z16 = jnp.zeros((ZCH, PW), jnp.float32)

    accp, cntp = _sc_agg_wide()(x, eidx, ones, z128, z16)

    b1l2 = b1l.reshape(1, F)
    w2 = jnp.concatenate([W2l, W2r, jnp.zeros((PW - 2, F), jnp.float32)], 0)

    h, p, cnt = _tc1(accp, cntp, x, W1l, b1l2, W1r, w2)

    acc2p = _sc_agg_narrow()(p, eidx, z16)

    out = _tc2(acc2p, cnt, p, b2l.reshape(1, 1))
    return (out, h)


# speedup vs baseline: 2.1832x; 2.1832x over previous
import functools

import jax
import jax.numpy as jnp
from jax import lax
from jax.experimental import pallas as pl
from jax.experimental.pallas import tpu as pltpu
from jax.experimental.pallas import tpu_sc as plsc

N = 10000
E = 320000
F = 128
PW = 16
NC, NS = 2, 16
NW = NC * NS
EPT = E // NW
CH = 128
NCH = 80
EPAD = NCH * CH - EPT
DL = NW * EPAD
NR = N
ZCH = 100
NZ = N // ZCH
ZPT = -(-NZ // NS)


def _mesh():
    return plsc.VectorSubcoreMesh(
        core_axis_name="c", subcore_axis_name="s", num_cores=NC, num_subcores=NS
    )


def _pipelined_agg(tab_hbm, eidx_hbm, wid, rows, ebs, acc_sh,
                   gsems, ssems, isems, extra_scatter=None, extra_wait=None):
    ebase = wid * (NCH * 2)

    for b in (0, 1):
        pltpu.sync_copy(eidx_hbm.at[pl.ds(ebase + 2 * b, 2)], ebs[b])
        pltpu.async_copy(tab_hbm.at[ebs[b].at[0]], rows[b], gsems[b])

    def step(g, carry):
        for b in (0, 1, 2, 3):
            c = 4 * g + b
            s = b % 2
            q2 = (b + 2) % 4
            pltpu.make_async_copy(tab_hbm.at[ebs[b].at[0]], rows[s],
                                  gsems[s]).wait()
            pltpu.async_copy(rows[s], acc_sh.at[ebs[b].at[1]], ssems[s],
                             add=True)
            if extra_scatter is not None:
                extra_scatter(s, ebs[b].at[1])

            @pl.when(c + 2 < NCH)
            def _():
                pltpu.async_copy(eidx_hbm.at[pl.ds(ebase + 2 * (c + 2), 2)],
                                 ebs[q2], isems[q2])

            pltpu.make_async_copy(rows[s], acc_sh.at[ebs[b].at[1]],
                                  ssems[s]).wait()
            if extra_wait is not None:
                extra_wait(s, ebs[b].at[1])

            @pl.when(c + 2 < NCH)
            def _():
                pltpu.make_async_copy(eidx_hbm.at[pl.ds(ebase + 2 * (c + 2),
                                                        2)],
                                      ebs[q2], isems[q2]).wait()
                pltpu.async_copy(tab_hbm.at[ebs[q2].at[0]], rows[s], gsems[s])

        return carry

    lax.fori_loop(0, NCH // 4, step, 0)


@functools.cache
def _sc_agg_wide():
    return pl.kernel(
        _sc_agg_wide_body,
        out_type=(
            jax.ShapeDtypeStruct((NC, N, F), jnp.float32),
            jax.ShapeDtypeStruct((NC, N, PW), jnp.float32),
        ),
        mesh=_mesh(),
        scratch_types=[
            pltpu.VMEM((2, CH), jnp.int32),
            pltpu.VMEM((2, CH), jnp.int32),
            pltpu.VMEM((2, CH), jnp.int32),
            pltpu.VMEM((2, CH), jnp.int32),
            pltpu.VMEM((CH, F), jnp.float32),
            pltpu.VMEM((CH, F), jnp.float32),
            pltpu.VMEM((CH, PW), jnp.float32),
            pltpu.VMEM((ZCH, PW), jnp.float32),
            pltpu.VMEM_SHARED((NR, F), jnp.float32),
            pltpu.VMEM_SHARED((NR, PW), jnp.float32),
            pltpu.SemaphoreType.DMA,
            pltpu.SemaphoreType.DMA,
            pltpu.SemaphoreType.DMA,
            pltpu.SemaphoreType.DMA,
            pltpu.SemaphoreType.DMA,
            pltpu.SemaphoreType.DMA,
            pltpu.SemaphoreType.DMA,
            pltpu.SemaphoreType.DMA,
            pltpu.SemaphoreType.DMA,
            pltpu.SemaphoreType.DMA,
        ],
        compiler_params=pltpu.CompilerParams(use_tc_tiling_on_sc=False),
    )


def _sc_agg_wide_body(x_hbm, eidx_hbm, ones_hbm, z128_hbm, z16_hbm,
                      acc_out, cnt_out,
                      eb0, eb1, eb2, eb3, rows0, rows1, ones_v, cb_v,
                      acc_sh, cnt_sh,
                      gsem0, gsem1, ssem0, ssem1,
                      isem0, isem1, isem2, isem3, osem0, osem1):
    cid = lax.axis_index("c")
    sid = lax.axis_index("s")
    wid = sid * NC + cid

    pltpu.sync_copy(z128_hbm, rows0.at[pl.ds(0, ZCH)])
    pltpu.sync_copy(z16_hbm, cb_v)
    pltpu.sync_copy(ones_hbm, ones_v)

    def zero_step(j, carry):
        k = sid + NS * j

        @pl.when(k < NZ)
        def _():
            pltpu.sync_copy(rows0.at[pl.ds(0, ZCH)],
                            acc_sh.at[pl.ds(k * ZCH, ZCH)])
            pltpu.sync_copy(cb_v, cnt_sh.at[pl.ds(k * ZCH, ZCH)])

        return carry

    lax.fori_loop(0, ZPT, zero_step, 0)
    plsc.subcore_barrier()
    osems = (osem0, osem1)

    def ones_scatter(s, dst_idx):
        pltpu.async_copy(ones_v, cnt_sh.at[dst_idx], osems[s], add=True)

    def ones_wait(s, dst_idx):
        pltpu.make_async_copy(ones_v, cnt_sh.at[dst_idx], osems[s]).wait()

    _pipelined_agg(x_hbm, eidx_hbm, wid, (rows0, rows1),
                   (eb0, eb1, eb2, eb3), acc_sh,
                   (gsem0, gsem1), (ssem0, ssem1),
                   (isem0, isem1, isem2, isem3),
                   extra_scatter=ones_scatter, extra_wait=ones_wait)
    plsc.subcore_barrier()

    def rb_step(j, carry):
        k = sid + NS * j

        @pl.when(k < NZ)
        def _():
            pltpu.sync_copy(acc_sh.at[pl.ds(k * ZCH, ZCH)],
                            rows0.at[pl.ds(0, ZCH)])
            pltpu.sync_copy(rows0.at[pl.ds(0, ZCH)],
                            acc_out.at[cid, pl.ds(k * ZCH, ZCH)])
            pltpu.sync_copy(cnt_sh.at[pl.ds(k * ZCH, ZCH)], cb_v)
            pltpu.sync_copy(cb_v, cnt_out.at[cid, pl.ds(k * ZCH, ZCH)])

        return carry

    lax.fori_loop(0, ZPT, rb_step, 0)


@functools.cache
def _sc_agg_narrow():
    return pl.kernel(
        _sc_agg_narrow_body,
        out_type=jax.ShapeDtypeStruct((NC, N, PW), jnp.float32),
        mesh=_mesh(),
        scratch_types=[
            pltpu.VMEM((2, CH), jnp.int32),
            pltpu.VMEM((2, CH), jnp.int32),
            pltpu.VMEM((2, CH), jnp.int32),
            pltpu.VMEM((2, CH), jnp.int32),
            pltpu.VMEM((CH, PW), jnp.float32),
            pltpu.VMEM((CH, PW), jnp.float32),
            pltpu.VMEM((ZCH, PW), jnp.float32),
            pltpu.VMEM_SHARED((NR, PW), jnp.float32),
            pltpu.SemaphoreType.DMA,
            pltpu.SemaphoreType.DMA,
            pltpu.SemaphoreType.DMA,
            pltpu.SemaphoreType.DMA,
            pltpu.SemaphoreType.DMA,
            pltpu.SemaphoreType.DMA,
            pltpu.SemaphoreType.DMA,
            pltpu.SemaphoreType.DMA,
        ],
        compiler_params=pltpu.CompilerParams(use_tc_tiling_on_sc=False),
    )


def _sc_agg_narrow_body(p_hbm, eidx_hbm, z16_hbm, acc_out,
                        eb0, eb1, eb2, eb3, rows0, rows1, cb_v, acc_sh,
                        gsem0, gsem1, ssem0, ssem1,
                        isem0, isem1, isem2, isem3):
    cid = lax.axis_index("c")
    sid = lax.axis_index("s")
    wid = sid * NC + cid

    pltpu.sync_copy(z16_hbm, cb_v)

    def zero_step(j, carry):
        k = sid + NS * j

        @pl.when(k < NZ)
        def _():
            pltpu.sync_copy(cb_v, acc_sh.at[pl.ds(k * ZCH, ZCH)])

        return carry

    lax.fori_loop(0, ZPT, zero_step, 0)
    plsc.subcore_barrier()
    _pipelined_agg(p_hbm, eidx_hbm, wid, (rows0, rows1),
                   (eb0, eb1, eb2, eb3), acc_sh,
                   (gsem0, gsem1), (ssem0, ssem1),
                   (isem0, isem1, isem2, isem3))
    plsc.subcore_barrier()

    def rb_step(j, carry):
        k = sid + NS * j

        @pl.when(k < NZ)
        def _():
            pltpu.sync_copy(acc_sh.at[pl.ds(k * ZCH, ZCH)], cb_v)
            pltpu.sync_copy(cb_v, acc_out.at[cid, pl.ds(k * ZCH, ZCH)])

        return carry

    lax.fori_loop(0, ZPT, rb_step, 0)


RBLK = 1000


def _tc1_body(accp, cntp, x, w1l, b1l, w1r, w2, h_out, p_out, cnt_out):
    a = accp[...]
    c = cntp[...]
    xb = x[...]
    row = RBLK * pl.program_id(0) + lax.broadcasted_iota(
        jnp.int32, (RBLK, 1), 0)
    dmask = (row < DL).astype(jnp.float32)
    acc = a[0] + a[1] - dmask * xb
    cnt = jnp.maximum(c[0, :, 0:1] + c[1, :, 0:1] - dmask, 1.0)
    agg = acc / cnt
    dn = (((1,), (1,)), ((), ()))
    h = (
        lax.dot_general(agg, w1l[...], dn, preferred_element_type=jnp.float32)
        + b1l[...]
        + lax.dot_general(xb, w1r[...], dn,
                          preferred_element_type=jnp.float32)
    )
    h = jnp.maximum(h, 0.0)
    h_out[...] = h
    p_out[...] = lax.dot_general(h, w2[...], dn,
                                 preferred_element_type=jnp.float32)
    cnt_out[...] = cnt


def _tc1(accp, cntp, x, w1l, b1l, w1r, w2, interpret=False):
    return pl.pallas_call(
        _tc1_body,
        grid=(N // RBLK,),
        in_specs=[
            pl.BlockSpec((NC, RBLK, F), lambda i: (0, i, 0)),
            pl.BlockSpec((NC, RBLK, PW), lambda i: (0, i, 0)),
            pl.BlockSpec((RBLK, F), lambda i: (i, 0)),
            pl.BlockSpec((F, F), lambda i: (0, 0)),
            pl.BlockSpec((1, F), lambda i: (0, 0)),
            pl.BlockSpec((F, F), lambda i: (0, 0)),
            pl.BlockSpec((PW, F), lambda i: (0, 0)),
        ],
        out_specs=[
            pl.BlockSpec((RBLK, F), lambda i: (i, 0)),
            pl.BlockSpec((RBLK, PW), lambda i: (i, 0)),
            pl.BlockSpec((RBLK, 1), lambda i: (i, 0)),
        ],
        out_shape=[
            jax.ShapeDtypeStruct((N, F), jnp.float32),
            jax.ShapeDtypeStruct((N, PW), jnp.float32),
            jax.ShapeDtypeStruct((N, 1), jnp.float32),
        ],
        interpret=interpret,
    )(accp, cntp, x, w1l, b1l, w1r, w2)


def _tc2_body(acc2p, cnt, p, b2l, out_ref):
    a = acc2p[...]
    pb = p[...]
    row = RBLK * pl.program_id(0) + lax.broadcasted_iota(
        jnp.int32, (RBLK, 1), 0)
    dmask = (row < DL).astype(jnp.float32)
    s = a[0, :, 0:1] + a[1, :, 0:1] - dmask * pb[:, 0:1]
    z = s / cnt[...] + b2l[0, 0] + pb[:, 1:2]
    out_ref[...] = jax.nn.sigmoid(z)


def _tc2(acc2p, cnt, p, b2l, interpret=False):
    return pl.pallas_call(
        _tc2_body,
        grid=(N // RBLK,),
        in_specs=[
            pl.BlockSpec((NC, RBLK, PW), lambda i: (0, i, 0)),
            pl.BlockSpec((RBLK, 1), lambda i: (i, 0)),
            pl.BlockSpec((RBLK, PW), lambda i: (i, 0)),
            pl.BlockSpec((1, 1), lambda i: (0, 0)),
        ],
        out_specs=pl.BlockSpec((RBLK, 1), lambda i: (i, 0)),
        out_shape=jax.ShapeDtypeStruct((N, 1), jnp.float32),
        interpret=interpret,
    )(acc2p, cnt, p, b2l)


def kernel(x, edge_index, W1l, b1l, W1r, W2l, b2l, W2r):
    ei = edge_index.astype(jnp.int32)
    dummy = jnp.arange(DL, dtype=jnp.int32).reshape(NW, EPAD)
    srcp = jnp.concatenate([ei[0].reshape(NW, EPT), dummy], axis=1)
    dstp = jnp.concatenate([ei[1].reshape(NW, EPT), dummy], axis=1)
    eidx = jnp.stack([srcp.reshape(NW, NCH, CH), dstp.reshape(NW, NCH, CH)],
                     axis=2).reshape(NW * NCH * 2, CH)
    ones = jnp.ones((CH, PW), jnp.float32)
    z128 = jnp.zeros((ZCH, F), jnp.float32)
    z16 = jnp.zeros((ZCH, PW), jnp.float32)

    accp, cntp = _sc_agg_wide()(x, eidx, ones, z128, z16)

    b1l2 = b1l.reshape(1, F)
    w2 = jnp.concatenate([W2l, W2r, jnp.zeros((PW - 2, F), jnp.float32)], 0)

    h, p, cnt = _tc1(accp, cntp, x, W1l, b1l2, W1r, w2)

    acc2p = _sc_agg_narrow()(p, eidx, z16)

    out = _tc2(acc2p, cnt, p, b2l.reshape(1, 1))
    return (out, h)


# speedup vs baseline: 2.2312x; 1.0220x over previous
import functools

import jax
import jax.numpy as jnp
from jax import lax
from jax.experimental import pallas as pl
from jax.experimental.pallas import tpu as pltpu
from jax.experimental.pallas import tpu_sc as plsc

N = 10000
E = 320000
F = 128
PW = 16
NC, NS = 2, 16
NW = NC * NS
EPT = E // NW
CH = 128
NCH = 80
EPAD = NCH * CH - EPT
DL = NW * EPAD
NR = N
ZCH = 100
NZ = N // ZCH
ZPT = -(-NZ // NS)


def _mesh():
    return plsc.VectorSubcoreMesh(
        core_axis_name="c", subcore_axis_name="s", num_cores=NC, num_subcores=NS
    )


def _pipelined_agg(tab_hbm, eidx_hbm, wid, rows, ebs, acc_sh,
                   gsems, ssems, isems, extra_scatter=None, extra_wait=None):
    ebase = wid * (NCH * 2)

    for b in (0, 1):
        pltpu.sync_copy(eidx_hbm.at[pl.ds(ebase + 2 * b, 2)], ebs[b])
        pltpu.async_copy(tab_hbm.at[ebs[b].at[0]], rows[b], gsems[b])

    def step(g, carry):
        for b in (0, 1, 2, 3):
            c = 4 * g + b
            s = b % 2
            q2 = (b + 2) % 4
            pltpu.make_async_copy(tab_hbm.at[ebs[b].at[0]], rows[s],
                                  gsems[s]).wait()
            pltpu.async_copy(rows[s], acc_sh.at[ebs[b].at[1]], ssems[s],
                             add=True)
            if extra_scatter is not None:
                extra_scatter(s, ebs[b].at[1])

            @pl.when(c + 2 < NCH)
            def _():
                pltpu.async_copy(eidx_hbm.at[pl.ds(ebase + 2 * (c + 2), 2)],
                                 ebs[q2], isems[q2])

            pltpu.make_async_copy(rows[s], acc_sh.at[ebs[b].at[1]],
                                  ssems[s]).wait()
            if extra_wait is not None:
                extra_wait(s, ebs[b].at[1])

            @pl.when(c + 2 < NCH)
            def _():
                pltpu.make_async_copy(eidx_hbm.at[pl.ds(ebase + 2 * (c + 2),
                                                        2)],
                                      ebs[q2], isems[q2]).wait()
                pltpu.async_copy(tab_hbm.at[ebs[q2].at[0]], rows[s], gsems[s])

        return carry

    lax.fori_loop(0, NCH // 4, step, 0)


@functools.cache
def _sc_agg_wide():
    return pl.kernel(
        _sc_agg_wide_body,
        out_type=(
            jax.ShapeDtypeStruct((NC, N, F), jnp.float32),
            jax.ShapeDtypeStruct((NC, N, PW), jnp.float32),
        ),
        mesh=_mesh(),
        scratch_types=[
            pltpu.VMEM((2, CH), jnp.int32),
            pltpu.VMEM((2, CH), jnp.int32),
            pltpu.VMEM((2, CH), jnp.int32),
            pltpu.VMEM((2, CH), jnp.int32),
            pltpu.VMEM((CH, F), jnp.float32),
            pltpu.VMEM((CH, F), jnp.float32),
            pltpu.VMEM((CH, PW), jnp.float32),
            pltpu.VMEM((ZCH, PW), jnp.float32),
            pltpu.VMEM_SHARED((NR, F), jnp.float32),
            pltpu.VMEM_SHARED((NR, PW), jnp.float32),
            pltpu.SemaphoreType.DMA,
            pltpu.SemaphoreType.DMA,
            pltpu.SemaphoreType.DMA,
            pltpu.SemaphoreType.DMA,
            pltpu.SemaphoreType.DMA,
            pltpu.SemaphoreType.DMA,
            pltpu.SemaphoreType.DMA,
            pltpu.SemaphoreType.DMA,
            pltpu.SemaphoreType.DMA,
            pltpu.SemaphoreType.DMA,
        ],
        compiler_params=pltpu.CompilerParams(use_tc_tiling_on_sc=False),
    )


def _sc_agg_wide_body(x_hbm, eidx_hbm, ones_hbm, z128_hbm, z16_hbm,
                      acc_out, cnt_out,
                      eb0, eb1, eb2, eb3, rows0, rows1, ones_v, cb_v,
                      acc_sh, cnt_sh,
                      gsem0, gsem1, ssem0, ssem1,
                      isem0, isem1, isem2, isem3, osem0, osem1):
    cid = lax.axis_index("c")
    sid = lax.axis_index("s")
    wid = sid * NC + cid

    pltpu.sync_copy(z128_hbm, rows0.at[pl.ds(0, ZCH)])
    pltpu.sync_copy(z16_hbm, cb_v)
    pltpu.sync_copy(ones_hbm, ones_v)

    def zero_step(j, carry):
        k = sid + NS * j

        @pl.when(k < NZ)
        def _():
            pltpu.sync_copy(rows0.at[pl.ds(0, ZCH)],
                            acc_sh.at[pl.ds(k * ZCH, ZCH)])
            pltpu.sync_copy(cb_v, cnt_sh.at[pl.ds(k * ZCH, ZCH)])

        return carry

    lax.fori_loop(0, ZPT, zero_step, 0)
    plsc.subcore_barrier()
    osems = (osem0, osem1)

    def ones_scatter(s, dst_idx):
        pltpu.async_copy(ones_v, cnt_sh.at[dst_idx], osems[s], add=True)

    def ones_wait(s, dst_idx):
        pltpu.make_async_copy(ones_v, cnt_sh.at[dst_idx], osems[s]).wait()

    _pipelined_agg(x_hbm, eidx_hbm, wid, (rows0, rows1),
                   (eb0, eb1, eb2, eb3), acc_sh,
                   (gsem0, gsem1), (ssem0, ssem1),
                   (isem0, isem1, isem2, isem3),
                   extra_scatter=ones_scatter, extra_wait=ones_wait)
    plsc.subcore_barrier()

    def rb_step(j, carry):
        k = sid + NS * j

        @pl.when(k < NZ)
        def _():
            pltpu.sync_copy(acc_sh.at[pl.ds(k * ZCH, ZCH)],
                            rows0.at[pl.ds(0, ZCH)])
            pltpu.sync_copy(rows0.at[pl.ds(0, ZCH)],
                            acc_out.at[cid, pl.ds(k * ZCH, ZCH)])
            pltpu.sync_copy(cnt_sh.at[pl.ds(k * ZCH, ZCH)], cb_v)
            pltpu.sync_copy(cb_v, cnt_out.at[cid, pl.ds(k * ZCH, ZCH)])

        return carry

    lax.fori_loop(0, ZPT, rb_step, 0)


@functools.cache
def _sc_agg_narrow():
    return pl.kernel(
        _sc_agg_narrow_body,
        out_type=jax.ShapeDtypeStruct((NC, N, PW), jnp.float32),
        mesh=_mesh(),
        scratch_types=[
            pltpu.VMEM((2, CH), jnp.int32),
            pltpu.VMEM((2, CH), jnp.int32),
            pltpu.VMEM((2, CH), jnp.int32),
            pltpu.VMEM((2, CH), jnp.int32),
            pltpu.VMEM((CH, PW), jnp.float32),
            pltpu.VMEM((CH, PW), jnp.float32),
            pltpu.VMEM((ZCH, PW), jnp.float32),
            pltpu.VMEM_SHARED((NR, PW), jnp.float32),
            pltpu.VMEM_SHARED((NR, PW), jnp.float32),
            pltpu.SemaphoreType.DMA,
            pltpu.SemaphoreType.DMA,
            pltpu.SemaphoreType.DMA,
            pltpu.SemaphoreType.DMA,
            pltpu.SemaphoreType.DMA,
            pltpu.SemaphoreType.DMA,
            pltpu.SemaphoreType.DMA,
            pltpu.SemaphoreType.DMA,
        ],
        compiler_params=pltpu.CompilerParams(use_tc_tiling_on_sc=False),
    )


def _sc_agg_narrow_body(p_hbm, eidx_hbm, z16_hbm, acc_out,
                        eb0, eb1, eb2, eb3, rows0, rows1, cb_v, acc_sh, p_sh,
                        gsem0, gsem1, ssem0, ssem1,
                        isem0, isem1, isem2, isem3):
    cid = lax.axis_index("c")
    sid = lax.axis_index("s")
    wid = sid * NC + cid

    pltpu.sync_copy(z16_hbm, cb_v)

    def zero_step(j, carry):
        k = sid + NS * j

        @pl.when(k < NZ)
        def _():
            pltpu.sync_copy(cb_v, acc_sh.at[pl.ds(k * ZCH, ZCH)])
            pltpu.sync_copy(p_hbm.at[pl.ds(k * ZCH, ZCH)],
                            rows0.at[pl.ds(0, ZCH)])
            pltpu.sync_copy(rows0.at[pl.ds(0, ZCH)],
                            p_sh.at[pl.ds(k * ZCH, ZCH)])

        return carry

    lax.fori_loop(0, ZPT, zero_step, 0)
    plsc.subcore_barrier()
    _pipelined_agg(p_sh, eidx_hbm, wid, (rows0, rows1),
                   (eb0, eb1, eb2, eb3), acc_sh,
                   (gsem0, gsem1), (ssem0, ssem1),
                   (isem0, isem1, isem2, isem3))
    plsc.subcore_barrier()

    def rb_step(j, carry):
        k = sid + NS * j

        @pl.when(k < NZ)
        def _():
            pltpu.sync_copy(acc_sh.at[pl.ds(k * ZCH, ZCH)], cb_v)
            pltpu.sync_copy(cb_v, acc_out.at[cid, pl.ds(k * ZCH, ZCH)])

        return carry

    lax.fori_loop(0, ZPT, rb_step, 0)


RBLK = 1000


def _tc1_body(accp, cntp, x, w1l, b1l, w1r, w2, h_out, p_out, cnt_out):
    a = accp[...]
    c = cntp[...]
    xb = x[...]
    row = RBLK * pl.program_id(0) + lax.broadcasted_iota(
        jnp.int32, (RBLK, 1), 0)
    dmask = (row < DL).astype(jnp.float32)
    acc = a[0] + a[1] - dmask * xb
    cnt = jnp.maximum(c[0, :, 0:1] + c[1, :, 0:1] - dmask, 1.0)
    agg = acc / cnt
    dn = (((1,), (1,)), ((), ()))
    h = (
        lax.dot_general(agg, w1l[...], dn, preferred_element_type=jnp.float32)
        + b1l[...]
        + lax.dot_general(xb, w1r[...], dn,
                          preferred_element_type=jnp.float32)
    )
    h = jnp.maximum(h, 0.0)
    h_out[...] = h
    p_out[...] = lax.dot_general(h, w2[...], dn,
                                 preferred_element_type=jnp.float32)
    cnt_out[...] = cnt


def _tc1(accp, cntp, x, w1l, b1l, w1r, w2, interpret=False):
    return pl.pallas_call(
        _tc1_body,
        grid=(N // RBLK,),
        in_specs=[
            pl.BlockSpec((NC, RBLK, F), lambda i: (0, i, 0)),
            pl.BlockSpec((NC, RBLK, PW), lambda i: (0, i, 0)),
            pl.BlockSpec((RBLK, F), lambda i: (i, 0)),
            pl.BlockSpec((F, F), lambda i: (0, 0)),
            pl.BlockSpec((1, F), lambda i: (0, 0)),
            pl.BlockSpec((F, F), lambda i: (0, 0)),
            pl.BlockSpec((PW, F), lambda i: (0, 0)),
        ],
        out_specs=[
            pl.BlockSpec((RBLK, F), lambda i: (i, 0)),
            pl.BlockSpec((RBLK, PW), lambda i: (i, 0)),
            pl.BlockSpec((RBLK, 1), lambda i: (i, 0)),
        ],
        out_shape=[
            jax.ShapeDtypeStruct((N, F), jnp.float32),
            jax.ShapeDtypeStruct((N, PW), jnp.float32),
            jax.ShapeDtypeStruct((N, 1), jnp.float32),
        ],
        interpret=interpret,
    )(accp, cntp, x, w1l, b1l, w1r, w2)


def _tc2_body(acc2p, cnt, p, b2l, out_ref):
    a = acc2p[...]
    pb = p[...]
    row = RBLK * pl.program_id(0) + lax.broadcasted_iota(
        jnp.int32, (RBLK, 1), 0)
    dmask = (row < DL).astype(jnp.float32)
    s = a[0, :, 0:1] + a[1, :, 0:1] - dmask * pb[:, 0:1]
    z = s / cnt[...] + b2l[0, 0] + pb[:, 1:2]
    out_ref[...] = jax.nn.sigmoid(z)


def _tc2(acc2p, cnt, p, b2l, interpret=False):
    return pl.pallas_call(
        _tc2_body,
        grid=(N // RBLK,),
        in_specs=[
            pl.BlockSpec((NC, RBLK, PW), lambda i: (0, i, 0)),
            pl.BlockSpec((RBLK, 1), lambda i: (i, 0)),
            pl.BlockSpec((RBLK, PW), lambda i: (i, 0)),
            pl.BlockSpec((1, 1), lambda i: (0, 0)),
        ],
        out_specs=pl.BlockSpec((RBLK, 1), lambda i: (i, 0)),
        out_shape=jax.ShapeDtypeStruct((N, 1), jnp.float32),
        interpret=interpret,
    )(acc2p, cnt, p, b2l)


def kernel(x, edge_index, W1l, b1l, W1r, W2l, b2l, W2r):
    ei = edge_index.astype(jnp.int32)
    dummy = jnp.arange(DL, dtype=jnp.int32).reshape(NW, EPAD)
    srcp = jnp.concatenate([ei[0].reshape(NW, EPT), dummy], axis=1)
    dstp = jnp.concatenate([ei[1].reshape(NW, EPT), dummy], axis=1)
    eidx = jnp.stack([srcp.reshape(NW, NCH, CH), dstp.reshape(NW, NCH, CH)],
                     axis=2).reshape(NW * NCH * 2, CH)
    ones = jnp.ones((CH, PW), jnp.float32)
    z128 = jnp.zeros((ZCH, F), jnp.float32)
    z16 = jnp.zeros((ZCH, PW), jnp.float32)

    accp, cntp = _sc_agg_wide()(x, eidx, ones, z128, z16)

    b1l2 = b1l.reshape(1, F)
    w2 = jnp.concatenate([W2l, W2r, jnp.zeros((PW - 2, F), jnp.float32)], 0)

    h, p, cnt = _tc1(accp, cntp, x, W1l, b1l2, W1r, w2)

    acc2p = _sc_agg_narrow()(p, eidx, z16)

    out = _tc2(acc2p, cnt, p, b2l.reshape(1, 1))
    return (out, h)


# speedup vs baseline: 2.2352x; 1.0018x over previous
import functools

import jax
import jax.numpy as jnp
from jax import lax
from jax.experimental import pallas as pl
from jax.experimental.pallas import tpu as pltpu
from jax.experimental.pallas import tpu_sc as plsc

N = 10000
E = 320000
F = 128
PW = 16
NC, NS = 2, 16
NW = NC * NS
EPT = E // NW
CH = 128
NCH = 80
EPAD = NCH * CH - EPT
DL = NW * EPAD
NR = N
ZCH = 100
NZ = N // ZCH
ZPT = -(-NZ // NS)


def _mesh():
    return plsc.VectorSubcoreMesh(
        core_axis_name="c", subcore_axis_name="s", num_cores=NC, num_subcores=NS
    )


def _pipelined_agg(tab_hbm, eidx_hbm, wid, rows, ebs, acc_sh,
                   gsems, ssems, isems, ones_v=None, cnt_sh=None, osems=None):
    ebase = wid * (NCH * 2)

    def ones_wait(b):
        pltpu.make_async_copy(ones_v, cnt_sh.at[ebs[b].at[1]],
                              osems[b]).wait()

    for b in (0, 1):
        pltpu.sync_copy(eidx_hbm.at[pl.ds(ebase + 2 * b, 2)], ebs[b])
        pltpu.async_copy(tab_hbm.at[ebs[b].at[0]], rows[b], gsems[b])

    def step(g, carry):
        for b in (0, 1, 2, 3):
            c = 4 * g + b
            s = b % 2
            q2 = (b + 2) % 4
            pltpu.make_async_copy(tab_hbm.at[ebs[b].at[0]], rows[s],
                                  gsems[s]).wait()
            pltpu.async_copy(rows[s], acc_sh.at[ebs[b].at[1]], ssems[s],
                             add=True)
            if osems is not None:
                pltpu.async_copy(ones_v, cnt_sh.at[ebs[b].at[1]], osems[b],
                                 add=True)

                @pl.when(c >= 2)
                def _():
                    ones_wait(q2)

            @pl.when(c + 2 < NCH)
            def _():
                pltpu.async_copy(eidx_hbm.at[pl.ds(ebase + 2 * (c + 2), 2)],
                                 ebs[q2], isems[q2])

            pltpu.make_async_copy(rows[s], acc_sh.at[ebs[b].at[1]],
                                  ssems[s]).wait()

            @pl.when(c + 2 < NCH)
            def _():
                pltpu.make_async_copy(eidx_hbm.at[pl.ds(ebase + 2 * (c + 2),
                                                        2)],
                                      ebs[q2], isems[q2]).wait()
                pltpu.async_copy(tab_hbm.at[ebs[q2].at[0]], rows[s], gsems[s])

        return carry

    lax.fori_loop(0, NCH // 4, step, 0)
    if osems is not None:
        ones_wait((NCH - 2) % 4)
        ones_wait((NCH - 1) % 4)


@functools.cache
def _sc_agg_wide():
    return pl.kernel(
        _sc_agg_wide_body,
        out_type=(
            jax.ShapeDtypeStruct((NC, N, F), jnp.float32),
            jax.ShapeDtypeStruct((NC, N, PW), jnp.float32),
        ),
        mesh=_mesh(),
        scratch_types=[
            pltpu.VMEM((2, CH), jnp.int32),
            pltpu.VMEM((2, CH), jnp.int32),
            pltpu.VMEM((2, CH), jnp.int32),
            pltpu.VMEM((2, CH), jnp.int32),
            pltpu.VMEM((CH, F), jnp.float32),
            pltpu.VMEM((CH, F), jnp.float32),
            pltpu.VMEM((CH, PW), jnp.float32),
            pltpu.VMEM((ZCH, PW), jnp.float32),
            pltpu.VMEM_SHARED((NR, F), jnp.float32),
            pltpu.VMEM_SHARED((NR, PW), jnp.float32),
            pltpu.SemaphoreType.DMA,
            pltpu.SemaphoreType.DMA,
            pltpu.SemaphoreType.DMA,
            pltpu.SemaphoreType.DMA,
            pltpu.SemaphoreType.DMA,
            pltpu.SemaphoreType.DMA,
            pltpu.SemaphoreType.DMA,
            pltpu.SemaphoreType.DMA,
            pltpu.SemaphoreType.DMA,
            pltpu.SemaphoreType.DMA,
            pltpu.SemaphoreType.DMA,
            pltpu.SemaphoreType.DMA,
        ],
        compiler_params=pltpu.CompilerParams(use_tc_tiling_on_sc=False),
    )


def _sc_agg_wide_body(x_hbm, eidx_hbm, ones_hbm, z128_hbm, z16_hbm,
                      acc_out, cnt_out,
                      eb0, eb1, eb2, eb3, rows0, rows1, ones_v, cb_v,
                      acc_sh, cnt_sh,
                      gsem0, gsem1, ssem0, ssem1,
                      isem0, isem1, isem2, isem3,
                      osem0, osem1, osem2, osem3):
    cid = lax.axis_index("c")
    sid = lax.axis_index("s")
    wid = sid * NC + cid

    pltpu.sync_copy(z128_hbm, rows0.at[pl.ds(0, ZCH)])
    pltpu.sync_copy(z16_hbm, cb_v)
    pltpu.sync_copy(ones_hbm, ones_v)

    def zero_step(j, carry):
        k = sid + NS * j

        @pl.when(k < NZ)
        def _():
            pltpu.sync_copy(rows0.at[pl.ds(0, ZCH)],
                            acc_sh.at[pl.ds(k * ZCH, ZCH)])
            pltpu.sync_copy(cb_v, cnt_sh.at[pl.ds(k * ZCH, ZCH)])

        return carry

    lax.fori_loop(0, ZPT, zero_step, 0)
    plsc.subcore_barrier()
    _pipelined_agg(x_hbm, eidx_hbm, wid, (rows0, rows1),
                   (eb0, eb1, eb2, eb3), acc_sh,
                   (gsem0, gsem1), (ssem0, ssem1),
                   (isem0, isem1, isem2, isem3),
                   ones_v=ones_v, cnt_sh=cnt_sh,
                   osems=(osem0, osem1, osem2, osem3))
    plsc.subcore_barrier()

    def rb_step(j, carry):
        k = sid + NS * j

        @pl.when(k < NZ)
        def _():
            pltpu.sync_copy(acc_sh.at[pl.ds(k * ZCH, ZCH)],
                            rows0.at[pl.ds(0, ZCH)])
            pltpu.sync_copy(rows0.at[pl.ds(0, ZCH)],
                            acc_out.at[cid, pl.ds(k * ZCH, ZCH)])
            pltpu.sync_copy(cnt_sh.at[pl.ds(k * ZCH, ZCH)], cb_v)
            pltpu.sync_copy(cb_v, cnt_out.at[cid, pl.ds(k * ZCH, ZCH)])

        return carry

    lax.fori_loop(0, ZPT, rb_step, 0)


@functools.cache
def _sc_agg_narrow():
    return pl.kernel(
        _sc_agg_narrow_body,
        out_type=jax.ShapeDtypeStruct((NC, N, PW), jnp.float32),
        mesh=_mesh(),
        scratch_types=[
            pltpu.VMEM((2, CH), jnp.int32),
            pltpu.VMEM((2, CH), jnp.int32),
            pltpu.VMEM((2, CH), jnp.int32),
            pltpu.VMEM((2, CH), jnp.int32),
            pltpu.VMEM((CH, PW), jnp.float32),
            pltpu.VMEM((CH, PW), jnp.float32),
            pltpu.VMEM((ZCH, PW), jnp.float32),
            pltpu.VMEM_SHARED((NR, PW), jnp.float32),
            pltpu.VMEM_SHARED((NR, PW), jnp.float32),
            pltpu.SemaphoreType.DMA,
            pltpu.SemaphoreType.DMA,
            pltpu.SemaphoreType.DMA,
            pltpu.SemaphoreType.DMA,
            pltpu.SemaphoreType.DMA,
            pltpu.SemaphoreType.DMA,
            pltpu.SemaphoreType.DMA,
            pltpu.SemaphoreType.DMA,
        ],
        compiler_params=pltpu.CompilerParams(use_tc_tiling_on_sc=False),
    )


def _sc_agg_narrow_body(p_hbm, eidx_hbm, z16_hbm, acc_out,
                        eb0, eb1, eb2, eb3, rows0, rows1, cb_v, acc_sh, p_sh,
                        gsem0, gsem1, ssem0, ssem1,
                        isem0, isem1, isem2, isem3):
    cid = lax.axis_index("c")
    sid = lax.axis_index("s")
    wid = sid * NC + cid

    pltpu.sync_copy(z16_hbm, cb_v)

    def zero_step(j, carry):
        k = sid + NS * j

        @pl.when(k < NZ)
        def _():
            pltpu.sync_copy(cb_v, acc_sh.at[pl.ds(k * ZCH, ZCH)])
            pltpu.sync_copy(p_hbm.at[pl.ds(k * ZCH, ZCH)],
                            rows0.at[pl.ds(0, ZCH)])
            pltpu.sync_copy(rows0.at[pl.ds(0, ZCH)],
                            p_sh.at[pl.ds(k * ZCH, ZCH)])

        return carry

    lax.fori_loop(0, ZPT, zero_step, 0)
    plsc.subcore_barrier()
    _pipelined_agg(p_sh, eidx_hbm, wid, (rows0, rows1),
                   (eb0, eb1, eb2, eb3), acc_sh,
                   (gsem0, gsem1), (ssem0, ssem1),
                   (isem0, isem1, isem2, isem3))
    plsc.subcore_barrier()

    def rb_step(j, carry):
        k = sid + NS * j

        @pl.when(k < NZ)
        def _():
            pltpu.sync_copy(acc_sh.at[pl.ds(k * ZCH, ZCH)], cb_v)
            pltpu.sync_copy(cb_v, acc_out.at[cid, pl.ds(k * ZCH, ZCH)])

        return carry

    lax.fori_loop(0, ZPT, rb_step, 0)


RBLK = 1000


def _tc1_body(accp, cntp, x, w1l, b1l, w1r, w2, h_out, p_out, cnt_out):
    a = accp[...]
    c = cntp[...]
    xb = x[...]
    row = RBLK * pl.program_id(0) + lax.broadcasted_iota(
        jnp.int32, (RBLK, 1), 0)
    dmask = (row < DL).astype(jnp.float32)
    acc = a[0] + a[1] - dmask * xb
    cnt = jnp.maximum(c[0, :, 0:1] + c[1, :, 0:1] - dmask, 1.0)
    agg = acc / cnt
    dn = (((1,), (1,)), ((), ()))
    h = (
        lax.dot_general(agg, w1l[...], dn, preferred_element_type=jnp.float32)
        + b1l[...]
        + lax.dot_general(xb, w1r[...], dn,
                          preferred_element_type=jnp.float32)
    )
    h = jnp.maximum(h, 0.0)
    h_out[...] = h
    p_out[...] = lax.dot_general(h, w2[...], dn,
                                 preferred_element_type=jnp.float32)
    cnt_out[...] = cnt


def _tc1(accp, cntp, x, w1l, b1l, w1r, w2, interpret=False):
    return pl.pallas_call(
        _tc1_body,
        grid=(N // RBLK,),
        in_specs=[
            pl.BlockSpec((NC, RBLK, F), lambda i: (0, i, 0)),
            pl.BlockSpec((NC, RBLK, PW), lambda i: (0, i, 0)),
            pl.BlockSpec((RBLK, F), lambda i: (i, 0)),
            pl.BlockSpec((F, F), lambda i: (0, 0)),
            pl.BlockSpec((1, F), lambda i: (0, 0)),
            pl.BlockSpec((F, F), lambda i: (0, 0)),
            pl.BlockSpec((PW, F), lambda i: (0, 0)),
        ],
        out_specs=[
            pl.BlockSpec((RBLK, F), lambda i: (i, 0)),
            pl.BlockSpec((RBLK, PW), lambda i: (i, 0)),
            pl.BlockSpec((RBLK, 1), lambda i: (i, 0)),
        ],
        out_shape=[
            jax.ShapeDtypeStruct((N, F), jnp.float32),
            jax.ShapeDtypeStruct((N, PW), jnp.float32),
            jax.ShapeDtypeStruct((N, 1), jnp.float32),
        ],
        interpret=interpret,
    )(accp, cntp, x, w1l, b1l, w1r, w2)


def _tc2_body(acc2p, cnt, p, b2l, out_ref):
    a = acc2p[...]
    pb = p[...]
    row = RBLK * pl.program_id(0) + lax.broadcasted_iota(
        jnp.int32, (RBLK, 1), 0)
    dmask = (row < DL).astype(jnp.float32)
    s = a[0, :, 0:1] + a[1, :, 0:1] - dmask * pb[:, 0:1]
    z = s / cnt[...] + b2l[0, 0] + pb[:, 1:2]
    out_ref[...] = jax.nn.sigmoid(z)


def _tc2(acc2p, cnt, p, b2l, interpret=False):
    return pl.pallas_call(
        _tc2_body,
        grid=(N // RBLK,),
        in_specs=[
            pl.BlockSpec((NC, RBLK, PW), lambda i: (0, i, 0)),
            pl.BlockSpec((RBLK, 1), lambda i: (i, 0)),
            pl.BlockSpec((RBLK, PW), lambda i: (i, 0)),
            pl.BlockSpec((1, 1), lambda i: (0, 0)),
        ],
        out_specs=pl.BlockSpec((RBLK, 1), lambda i: (i, 0)),
        out_shape=jax.ShapeDtypeStruct((N, 1), jnp.float32),
        interpret=interpret,
    )(acc2p, cnt, p, b2l)


def kernel(x, edge_index, W1l, b1l, W1r, W2l, b2l, W2r):
    ei = edge_index.astype(jnp.int32)
    dummy = jnp.arange(DL, dtype=jnp.int32).reshape(NW, EPAD)
    srcp = jnp.concatenate([ei[0].reshape(NW, EPT), dummy], axis=1)
    dstp = jnp.concatenate([ei[1].reshape(NW, EPT), dummy], axis=1)
    eidx = jnp.stack([srcp.reshape(NW, NCH, CH), dstp.reshape(NW, NCH, CH)],
                     axis=2).reshape(NW * NCH * 2, CH)
    ones = jnp.ones((CH, PW), jnp.float32)
    z128 = jnp.zeros((ZCH, F), jnp.float32)
    z16 = jnp.zeros((ZCH, PW), jnp.float32)

    accp, cntp = _sc_agg_wide()(x, eidx, ones, z128, z16)

    b1l2 = b1l.reshape(1, F)
    w2 = jnp.concatenate([W2l, W2r, jnp.zeros((PW - 2, F), jnp.float32)], 0)

    h, p, cnt = _tc1(accp, cntp, x, W1l, b1l2, W1r, w2)

    acc2p = _sc_agg_narrow()(p, eidx, z16)

    out = _tc2(acc2p, cnt, p, b2l.reshape(1, 1))
    return (out, h)


# speedup vs baseline: 2.2769x; 1.0186x over previous
import functools

import jax
import jax.numpy as jnp
from jax import lax
from jax.experimental import pallas as pl
from jax.experimental.pallas import tpu as pltpu
from jax.experimental.pallas import tpu_sc as plsc

N = 10000
E = 320000
F = 128
PW = 16
NC, NS = 2, 16
NW = NC * NS
EPT = E // NW
CH = 128
NCH = 80
EPAD = NCH * CH - EPT
DL = NW * EPAD
NR = N
ZCH = 100
NZ = N // ZCH
ZPT = -(-NZ // NS)


def _mesh():
    return plsc.VectorSubcoreMesh(
        core_axis_name="c", subcore_axis_name="s", num_cores=NC, num_subcores=NS
    )


def _pipelined_agg(tab_hbm, eidx_hbm, wid, rows, ebs, acc_sh,
                   gsems, ssems, isems, ones_v=None, cnt_sh=None, osems=None):
    ebase = wid * (NCH * 2)

    def ones_wait(b):
        pltpu.make_async_copy(ones_v, cnt_sh.at[ebs[b].at[1]],
                              osems[b]).wait()

    for b in (0, 1):
        pltpu.sync_copy(eidx_hbm.at[pl.ds(ebase + 2 * b, 2)], ebs[b])
        pltpu.async_copy(tab_hbm.at[ebs[b].at[0]], rows[b], gsems[b])

    def step(g, carry):
        for b in (0, 1, 2, 3):
            c = 4 * g + b
            s = b % 2
            q2 = (b + 2) % 4
            pltpu.make_async_copy(tab_hbm.at[ebs[b].at[0]], rows[s],
                                  gsems[s]).wait()
            pltpu.async_copy(rows[s], acc_sh.at[ebs[b].at[1]], ssems[s],
                             add=True)
            if osems is not None:
                pltpu.async_copy(ones_v, cnt_sh.at[ebs[b].at[1]], osems[b],
                                 add=True)

                @pl.when(c >= 2)
                def _():
                    ones_wait(q2)

            @pl.when(c + 2 < NCH)
            def _():
                pltpu.async_copy(eidx_hbm.at[pl.ds(ebase + 2 * (c + 2), 2)],
                                 ebs[q2], isems[q2])

            pltpu.make_async_copy(rows[s], acc_sh.at[ebs[b].at[1]],
                                  ssems[s]).wait()

            @pl.when(c + 2 < NCH)
            def _():
                pltpu.make_async_copy(eidx_hbm.at[pl.ds(ebase + 2 * (c + 2),
                                                        2)],
                                      ebs[q2], isems[q2]).wait()
                pltpu.async_copy(tab_hbm.at[ebs[q2].at[0]], rows[s], gsems[s])

        return carry

    lax.fori_loop(0, NCH // 4, step, 0)
    if osems is not None:
        ones_wait((NCH - 2) % 4)
        ones_wait((NCH - 1) % 4)


@functools.cache
def _sc_agg_wide():
    return pl.kernel(
        _sc_agg_wide_body,
        out_type=(
            jax.ShapeDtypeStruct((NC, N, F), jnp.float32),
            jax.ShapeDtypeStruct((NC, N, PW), jnp.float32),
        ),
        mesh=_mesh(),
        scratch_types=[
            pltpu.VMEM((2, CH), jnp.int32),
            pltpu.VMEM((2, CH), jnp.int32),
            pltpu.VMEM((2, CH), jnp.int32),
            pltpu.VMEM((2, CH), jnp.int32),
            pltpu.VMEM((CH, F), jnp.float32),
            pltpu.VMEM((CH, F), jnp.float32),
            pltpu.VMEM((CH, PW), jnp.float32),
            pltpu.VMEM((ZCH, PW), jnp.float32),
            pltpu.VMEM_SHARED((NR, F), jnp.float32),
            pltpu.VMEM_SHARED((NR, PW), jnp.float32),
            pltpu.SemaphoreType.DMA,
            pltpu.SemaphoreType.DMA,
            pltpu.SemaphoreType.DMA,
            pltpu.SemaphoreType.DMA,
            pltpu.SemaphoreType.DMA,
            pltpu.SemaphoreType.DMA,
            pltpu.SemaphoreType.DMA,
            pltpu.SemaphoreType.DMA,
            pltpu.SemaphoreType.DMA,
            pltpu.SemaphoreType.DMA,
            pltpu.SemaphoreType.DMA,
            pltpu.SemaphoreType.DMA,
        ],
        compiler_params=pltpu.CompilerParams(use_tc_tiling_on_sc=False),
    )


def _sc_agg_wide_body(x_hbm, eidx_hbm, ones_hbm, z128_hbm, z16_hbm,
                      acc_out, cnt_out,
                      eb0, eb1, eb2, eb3, rows0, rows1, ones_v, cb_v,
                      acc_sh, cnt_sh,
                      gsem0, gsem1, ssem0, ssem1,
                      isem0, isem1, isem2, isem3,
                      osem0, osem1, osem2, osem3):
    cid = lax.axis_index("c")
    sid = lax.axis_index("s")
    wid = sid * NC + cid

    pltpu.sync_copy(z128_hbm, rows0.at[pl.ds(0, ZCH)])
    pltpu.sync_copy(z16_hbm, cb_v)
    pltpu.sync_copy(ones_hbm, ones_v)

    def zero_step(j, carry):
        k = sid + NS * j

        @pl.when(k < NZ)
        def _():
            pltpu.sync_copy(rows0.at[pl.ds(0, ZCH)],
                            acc_sh.at[pl.ds(k * ZCH, ZCH)])
            pltpu.sync_copy(cb_v, cnt_sh.at[pl.ds(k * ZCH, ZCH)])

        return carry

    lax.fori_loop(0, ZPT, zero_step, 0)
    plsc.subcore_barrier()
    _pipelined_agg(x_hbm, eidx_hbm, wid, (rows0, rows1),
                   (eb0, eb1, eb2, eb3), acc_sh,
                   (gsem0, gsem1), (ssem0, ssem1),
                   (isem0, isem1, isem2, isem3),
                   ones_v=ones_v, cnt_sh=cnt_sh,
                   osems=(osem0, osem1, osem2, osem3))
    plsc.subcore_barrier()

    def rb_step(j, carry):
        k = sid + NS * j

        @pl.when(k < NZ)
        def _():
            pltpu.sync_copy(acc_sh.at[pl.ds(k * ZCH, ZCH)],
                            rows0.at[pl.ds(0, ZCH)])
            pltpu.sync_copy(rows0.at[pl.ds(0, ZCH)],
                            acc_out.at[cid, pl.ds(k * ZCH, ZCH)])
            pltpu.sync_copy(cnt_sh.at[pl.ds(k * ZCH, ZCH)], cb_v)
            pltpu.sync_copy(cb_v, cnt_out.at[cid, pl.ds(k * ZCH, ZCH)])

        return carry

    lax.fori_loop(0, ZPT, rb_step, 0)


@functools.cache
def _sc_agg_narrow():
    return pl.kernel(
        _sc_agg_narrow_body,
        out_type=jax.ShapeDtypeStruct((NC, N, PW), jnp.float32),
        mesh=_mesh(),
        scratch_types=[
            pltpu.VMEM((2, CH), jnp.int32),
            pltpu.VMEM((2, CH), jnp.int32),
            pltpu.VMEM((2, CH), jnp.int32),
            pltpu.VMEM((2, CH), jnp.int32),
            pltpu.VMEM((CH, PW), jnp.float32),
            pltpu.VMEM((CH, PW), jnp.float32),
            pltpu.VMEM((ZCH, PW), jnp.float32),
            pltpu.VMEM_SHARED((NR, PW), jnp.float32),
            pltpu.VMEM_SHARED((NR, PW), jnp.float32),
            pltpu.SemaphoreType.DMA,
            pltpu.SemaphoreType.DMA,
            pltpu.SemaphoreType.DMA,
            pltpu.SemaphoreType.DMA,
            pltpu.SemaphoreType.DMA,
            pltpu.SemaphoreType.DMA,
            pltpu.SemaphoreType.DMA,
            pltpu.SemaphoreType.DMA,
        ],
        compiler_params=pltpu.CompilerParams(use_tc_tiling_on_sc=False),
    )


def _sc_agg_narrow_body(p_hbm, eidx_hbm, z16_hbm, acc_out,
                        eb0, eb1, eb2, eb3, rows0, rows1, cb_v, acc_sh, p_sh,
                        gsem0, gsem1, ssem0, ssem1,
                        isem0, isem1, isem2, isem3):
    cid = lax.axis_index("c")
    sid = lax.axis_index("s")
    wid = sid * NC + cid

    pltpu.sync_copy(z16_hbm, cb_v)

    def zero_step(j, carry):
        k = sid + NS * j

        @pl.when(k < NZ)
        def _():
            pltpu.sync_copy(cb_v, acc_sh.at[pl.ds(k * ZCH, ZCH)])
            pltpu.sync_copy(p_hbm.at[pl.ds(k * ZCH, ZCH)],
                            rows0.at[pl.ds(0, ZCH)])
            pltpu.sync_copy(rows0.at[pl.ds(0, ZCH)],
                            p_sh.at[pl.ds(k * ZCH, ZCH)])

        return carry

    lax.fori_loop(0, ZPT, zero_step, 0)
    plsc.subcore_barrier()
    _pipelined_agg(p_sh, eidx_hbm, wid, (rows0, rows1),
                   (eb0, eb1, eb2, eb3), acc_sh,
                   (gsem0, gsem1), (ssem0, ssem1),
                   (isem0, isem1, isem2, isem3))
    plsc.subcore_barrier()

    def rb_step(j, carry):
        k = sid + NS * j

        @pl.when(k < NZ)
        def _():
            pltpu.sync_copy(acc_sh.at[pl.ds(k * ZCH, ZCH)], cb_v)
            pltpu.sync_copy(cb_v, acc_out.at[cid, pl.ds(k * ZCH, ZCH)])

        return carry

    lax.fori_loop(0, ZPT, rb_step, 0)


RBLK = 2000


def _tc1_body(accp, cntp, x, w1l, b1l, w1r, w2, h_out, p_out, cnt_out):
    a = accp[...]
    c = cntp[...]
    xb = x[...]
    row = RBLK * pl.program_id(0) + lax.broadcasted_iota(
        jnp.int32, (RBLK, 1), 0)
    dmask = (row < DL).astype(jnp.float32)
    acc = a[0] + a[1] - dmask * xb
    cnt = jnp.maximum(c[0, :, 0:1] + c[1, :, 0:1] - dmask, 1.0)
    agg = acc / cnt
    dn = (((1,), (1,)), ((), ()))
    h = (
        lax.dot_general(agg, w1l[...], dn, preferred_element_type=jnp.float32)
        + b1l[...]
        + lax.dot_general(xb, w1r[...], dn,
                          preferred_element_type=jnp.float32)
    )
    h = jnp.maximum(h, 0.0)
    h_out[...] = h
    p_out[...] = lax.dot_general(h, w2[...], dn,
                                 preferred_element_type=jnp.float32)
    cnt_out[...] = cnt


def _tc1(accp, cntp, x, w1l, b1l, w1r, w2, interpret=False):
    return pl.pallas_call(
        _tc1_body,
        grid=(N // RBLK,),
        in_specs=[
            pl.BlockSpec((NC, RBLK, F), lambda i: (0, i, 0)),
            pl.BlockSpec((NC, RBLK, PW), lambda i: (0, i, 0)),
            pl.BlockSpec((RBLK, F), lambda i: (i, 0)),
            pl.BlockSpec((F, F), lambda i: (0, 0)),
            pl.BlockSpec((1, F), lambda i: (0, 0)),
            pl.BlockSpec((F, F), lambda i: (0, 0)),
            pl.BlockSpec((PW, F), lambda i: (0, 0)),
        ],
        out_specs=[
            pl.BlockSpec((RBLK, F), lambda i: (i, 0)),
            pl.BlockSpec((RBLK, PW), lambda i: (i, 0)),
            pl.BlockSpec((RBLK, 1), lambda i: (i, 0)),
        ],
        out_shape=[
            jax.ShapeDtypeStruct((N, F), jnp.float32),
            jax.ShapeDtypeStruct((N, PW), jnp.float32),
            jax.ShapeDtypeStruct((N, 1), jnp.float32),
        ],
        interpret=interpret,
    )(accp, cntp, x, w1l, b1l, w1r, w2)


def _tc2_body(acc2p, cnt, p, b2l, out_ref):
    a = acc2p[...]
    pb = p[...]
    row = RBLK * pl.program_id(0) + lax.broadcasted_iota(
        jnp.int32, (RBLK, 1), 0)
    dmask = (row < DL).astype(jnp.float32)
    s = a[0, :, 0:1] + a[1, :, 0:1] - dmask * pb[:, 0:1]
    z = s / cnt[...] + b2l[0, 0] + pb[:, 1:2]
    out_ref[...] = jax.nn.sigmoid(z)


def _tc2(acc2p, cnt, p, b2l, interpret=False):
    return pl.pallas_call(
        _tc2_body,
        grid=(N // RBLK,),
        in_specs=[
            pl.BlockSpec((NC, RBLK, PW), lambda i: (0, i, 0)),
            pl.BlockSpec((RBLK, 1), lambda i: (i, 0)),
            pl.BlockSpec((RBLK, PW), lambda i: (i, 0)),
            pl.BlockSpec((1, 1), lambda i: (0, 0)),
        ],
        out_specs=pl.BlockSpec((RBLK, 1), lambda i: (i, 0)),
        out_shape=jax.ShapeDtypeStruct((N, 1), jnp.float32),
        interpret=interpret,
    )(acc2p, cnt, p, b2l)


def kernel(x, edge_index, W1l, b1l, W1r, W2l, b2l, W2r):
    ei = edge_index.astype(jnp.int32)
    dummy = jnp.arange(DL, dtype=jnp.int32).reshape(NW, EPAD)
    srcp = jnp.concatenate([ei[0].reshape(NW, EPT), dummy], axis=1)
    dstp = jnp.concatenate([ei[1].reshape(NW, EPT), dummy], axis=1)
    eidx = jnp.stack([srcp.reshape(NW, NCH, CH), dstp.reshape(NW, NCH, CH)],
                     axis=2).reshape(NW * NCH * 2, CH)
    ones = jnp.ones((CH, PW), jnp.float32)
    z128 = jnp.zeros((ZCH, F), jnp.float32)
    z16 = jnp.zeros((ZCH, PW), jnp.float32)

    accp, cntp = _sc_agg_wide()(x, eidx, ones, z128, z16)

    b1l2 = b1l.reshape(1, F)
    w2 = jnp.concatenate([W2l, W2r, jnp.zeros((PW - 2, F), jnp.float32)], 0)

    h, p, cnt = _tc1(accp, cntp, x, W1l, b1l2, W1r, w2)

    acc2p = _sc_agg_narrow()(p, eidx, z16)

    out = _tc2(acc2p, cnt, p, b2l.reshape(1, 1))
    return (out, h)
